# Initial kernel scaffold; baseline (speedup 1.0000x reference)
import functools

import jax
import jax.numpy as jnp
from jax import lax
from jax.experimental import pallas as pl
from jax.experimental.pallas import tpu as pltpu
from jax.experimental.pallas import tpu_sc as plsc

L = 16
NC = 2
NS = 16
NW = NC * NS
CHUNK = 2048
D = 5


def _sc_body(n_per_w, n_chunks, x_hbm, tab_hbm, out_hbm, tab_v, x_v, out_v):
  wid = lax.axis_index("s") * NC + lax.axis_index("c")
  base = wid * n_per_w

  pltpu.sync_copy(tab_hbm, tab_v)

  splats = [
      [plsc.load_gather(tab_v, [jnp.full((L,), v * D + k, jnp.int32)])
       for k in range(D)]
      for v in range(3)
  ]
  iota = lax.iota(jnp.int32, L)
  sidx = [iota * D + k for k in range(D)]

  def chunk_body(g, carry):
    off = base + g * CHUNK
    pltpu.sync_copy(x_hbm.at[pl.ds(off, CHUNK)], x_v)

    def it_body(i, c):
      xv = x_v[pl.ds(i * L, L)]
      m0 = xv == 0
      m1 = xv == 1
      ob = jnp.full((L,), i * (L * D), jnp.int32)
      for k in range(D):
        val = jnp.where(m0, splats[0][k],
                        jnp.where(m1, splats[1][k], splats[2][k]))
        plsc.store_scatter(out_v, [sidx[k] + ob], val)
      return c

    lax.fori_loop(0, CHUNK // L, it_body, 0)
    pltpu.sync_copy(out_v, out_hbm.at[pl.ds(off * D, CHUNK * D)])
    return carry

  lax.fori_loop(0, n_chunks, chunk_body, 0)


def kernel(x, table):
  B, S = x.shape
  n = B * S
  assert n % (NW * CHUNK) == 0
  n_per_w = n // NW
  n_chunks = n_per_w // CHUNK

  x_flat = x.reshape(n).astype(jnp.int32)
  tab16 = jnp.pad(table.reshape(-1), (0, L - 3 * D))

  mesh = plsc.VectorSubcoreMesh(core_axis_name="c", subcore_axis_name="s")
  out = pl.kernel(
      functools.partial(_sc_body, n_per_w, n_chunks),
      out_type=jax.ShapeDtypeStruct((n * D,), jnp.float32),
      mesh=mesh,
      scratch_types=[
          pltpu.VMEM((L,), jnp.float32),
          pltpu.VMEM((CHUNK,), jnp.int32),
          pltpu.VMEM((CHUNK * D,), jnp.float32),
      ],
  )(x_flat, tab16)
  return out.reshape(B, S, D)


# speedup vs baseline: 5.0434x; 5.0434x over previous
import functools

import jax
import jax.numpy as jnp
import numpy as np
from jax import lax
from jax.experimental import pallas as pl
from jax.experimental.pallas import tpu as pltpu
from jax.experimental.pallas import tpu_sc as plsc

L = 16
NC = 2
NS = 16
NW = NC * NS
CHUNK = 2048
D = 5

_POS = np.arange(L * D, dtype=np.int32).reshape(D, L)
_JPAT = _POS // D
_KPAT = _POS % D


def _sc_body(n_per_w, n_chunks, x_hbm, jpat_hbm, tpat_hbm, out_hbm,
             jpat_v, tpat_v, x_v, out_v):
  wid = lax.axis_index("s") * NC + lax.axis_index("c")
  base = wid * n_per_w

  pltpu.sync_copy(jpat_hbm, jpat_v)
  pltpu.sync_copy(tpat_hbm, tpat_v)
  jpat = [jpat_v[pl.ds(p * L, L)] for p in range(D)]
  tpats = [
      [tpat_v[pl.ds((v * D + p) * L, L)] for p in range(D)]
      for v in range(3)
  ]

  def chunk_body(g, carry):
    off = base + g * CHUNK
    pltpu.sync_copy(x_hbm.at[pl.ds(off, CHUNK)], x_v)

    def it_body(i, c):
      xv = x_v[pl.ds(i * L, L)]
      for p in range(D):
        xe = xv.at[jpat[p]].get(mode="promise_in_bounds")
        m0 = xe == 0
        m1 = xe == 1
        val = jnp.where(m0, tpats[0][p],
                        jnp.where(m1, tpats[1][p], tpats[2][p]))
        out_v[pl.ds(i * (L * D) + p * L, L)] = val
      return c

    lax.fori_loop(0, CHUNK // L, it_body, 0)
    pltpu.sync_copy(out_v, out_hbm.at[pl.ds(off * D, CHUNK * D)])
    return carry

  lax.fori_loop(0, n_chunks, chunk_body, 0)


def kernel(x, table):
  B, S = x.shape
  n = B * S
  assert n % (NW * CHUNK) == 0
  n_per_w = n // NW
  n_chunks = n_per_w // CHUNK

  x_flat = x.reshape(n).astype(jnp.int32)
  jpat = jnp.asarray(_JPAT.reshape(-1))
  tpat = table[:, _KPAT].reshape(-1)

  mesh = plsc.VectorSubcoreMesh(core_axis_name="c", subcore_axis_name="s")
  out = pl.kernel(
      functools.partial(_sc_body, n_per_w, n_chunks),
      out_type=jax.ShapeDtypeStruct((n * D,), jnp.float32),
      mesh=mesh,
      scratch_types=[
          pltpu.VMEM((D * L,), jnp.int32),
          pltpu.VMEM((3 * D * L,), jnp.float32),
          pltpu.VMEM((CHUNK,), jnp.int32),
          pltpu.VMEM((CHUNK * D,), jnp.float32),
      ],
  )(x_flat, jpat, tpat)
  return out.reshape(B, S, D)


# speedup vs baseline: 7.3884x; 1.4650x over previous
import functools

import jax
import jax.numpy as jnp
from jax import lax
from jax.experimental import pallas as pl
from jax.experimental.pallas import tpu as pltpu
from jax.experimental.pallas import tpu_sc as plsc

L = 16
NC = 2
NS = 16
NW = NC * NS
CHUNK = 800
D = 5


def _sc_body(n, x_hbm, tab_hbm, out_hbm, tab_v, x_v, int_v):
  wid = lax.axis_index("s") * NC + lax.axis_index("c")
  n_per_w = n // NW
  n_chunks = n_per_w // CHUNK
  out2 = out_hbm.reshape(n, D)

  pltpu.sync_copy(tab_hbm, tab_v)
  splats = [[tab_v[pl.ds(v * L, L)].at[jnp.full((L,), k, jnp.int32)].get(
      mode="promise_in_bounds") for k in range(D)] for v in range(3)]
  iota = lax.iota(jnp.int32, L)

  def chunk_body(g, carry):
    off = wid * n_per_w + g * CHUNK
    pltpu.sync_copy(x_hbm.at[pl.ds(off, CHUNK)], x_v)

    def it_body(i, c):
      xv = x_v[pl.ds(i * L, L)]
      m0 = xv == 0
      m1 = xv == 1
      rows = iota + i * L
      for k in range(D):
        val = jnp.where(m0, splats[0][k],
                        jnp.where(m1, splats[1][k], splats[2][k]))
        plsc.store_scatter(int_v, [rows, jnp.full((L,), k, jnp.int32)], val)
      return c

    lax.fori_loop(0, CHUNK // L, it_body, 0)
    pltpu.sync_copy(int_v, out2.at[pl.ds(off, CHUNK)])
    return carry

  lax.fori_loop(0, n_chunks, chunk_body, 0)


def kernel(x, table):
  B, S = x.shape
  n = B * S
  assert n % (NW * CHUNK) == 0

  x_flat = x.reshape(n).astype(jnp.int32)
  tab_pad = jnp.pad(table, ((0, 0), (0, L - D))).reshape(-1)

  mesh = plsc.VectorSubcoreMesh(core_axis_name="c", subcore_axis_name="s")
  out = pl.kernel(
      functools.partial(_sc_body, n),
      out_type=jax.ShapeDtypeStruct((B, S, D), jnp.float32),
      mesh=mesh,
      compiler_params=pltpu.CompilerParams(needs_layout_passes=False),
      scratch_types=[
          pltpu.VMEM((3 * L,), jnp.float32),
          pltpu.VMEM((CHUNK,), jnp.int32),
          pltpu.VMEM((CHUNK, D), jnp.float32),
      ],
  )(x_flat, tab_pad)
  return out


# speedup vs baseline: 239.3729x; 32.3984x over previous
import functools

import jax
import jax.numpy as jnp
from jax import lax
from jax.experimental import pallas as pl
from jax.experimental.pallas import tpu as pltpu
from jax.experimental.pallas import tpu_sc as plsc

L = 16
NC = 2
NS = 16
NW = NC * NS
JB = 8
D = 5


def _sc_body(B, S, x_hbm, tab_hbm, out_hbm, tab_v, x_v, out_v,
             sin0, sin1, sout0, sout1):
  wid = lax.axis_index("s") * NC + lax.axis_index("c")
  ib = B // NW
  i0 = wid * ib
  n_chunks = S // JB
  sins = (sin0, sin1)
  souts = (sout0, sout1)

  def in_cp(g, b):
    return pltpu.make_async_copy(
        x_hbm.at[pl.ds(g * JB, JB), pl.ds(i0, ib)], x_v.at[b], sins[b])

  in_cp(0, 0).start()

  pltpu.sync_copy(tab_hbm, tab_v)
  splats = [[tab_v[pl.ds(v * L, L)].at[jnp.full((L,), k, jnp.int32)].get(
      mode="promise_in_bounds") for k in range(D)] for v in range(3)]

  def out_cp(g, b):
    return pltpu.make_async_copy(
        out_v.at[b], out_hbm.at[:, pl.ds(g * JB, JB), pl.ds(i0, ib)],
        souts[b])

  def compute(b):
    def it_body(l, c):
      for j in range(JB):
        xv = x_v[b, j, pl.ds(l * L, L)]
        m0 = xv == 0
        m1 = xv == 1
        for k in range(D):
          val = jnp.where(m0, splats[0][k],
                          jnp.where(m1, splats[1][k], splats[2][k]))
          out_v[b, k, j, pl.ds(l * L, L)] = val
      return c

    lax.fori_loop(0, ib // L, it_body, 0)

  def pair_body(p, carry):
    for b in range(2):
      g = 2 * p + b

      @pl.when(g + 1 < n_chunks)
      def _():
        in_cp(g + 1, 1 - b).start()

      in_cp(g, b).wait()

      @pl.when(g >= 2)
      def _():
        out_cp(g - 2, b).wait()

      compute(b)
      out_cp(g, b).start()
    return carry

  lax.fori_loop(0, n_chunks // 2, pair_body, 0)

  if n_chunks % 2:
    g = n_chunks - 1
    in_cp(g, 0).wait()
    out_cp(g - 2, 0).wait()
    compute(0)
    out_cp(g, 0).start()
  out_cp(n_chunks - 2, (n_chunks - 2) % 2).wait()
  out_cp(n_chunks - 1, (n_chunks - 1) % 2).wait()


def kernel(x, table):
  B, S = x.shape
  assert B % (NW * L) == 0 and S % JB == 0
  ib = B // NW

  x_t = x.astype(jnp.int32).T
  tab_pad = jnp.pad(table, ((0, 0), (0, L - D))).reshape(-1)

  mesh = plsc.VectorSubcoreMesh(core_axis_name="c", subcore_axis_name="s")
  out_t = pl.kernel(
      functools.partial(_sc_body, B, S),
      out_type=jax.ShapeDtypeStruct((D, S, B), jnp.float32),
      mesh=mesh,
      scratch_types=[
          pltpu.VMEM((3 * L,), jnp.float32),
          pltpu.VMEM((2, JB, ib), jnp.int32),
          pltpu.VMEM((2, D, JB, ib), jnp.float32),
          pltpu.SemaphoreType.DMA,
          pltpu.SemaphoreType.DMA,
          pltpu.SemaphoreType.DMA,
          pltpu.SemaphoreType.DMA,
      ],
  )(x_t, tab_pad)
  return jnp.transpose(out_t, (2, 1, 0))
